# baseline (speedup 1.0000x reference)
import functools

import jax
import jax.numpy as jnp
from jax import lax
from jax.experimental import pallas as pl
from jax.experimental.pallas import tpu as pltpu
from jax.experimental.pallas import tpu_sc as plsc

_NC = 2
_NS = 16
_NW = _NC * _NS
_L = 16

_NB = 4


@functools.lru_cache(maxsize=None)
def _build_transpose(v, d, dpad):
    slab = 128
    nfull = v // slab
    tail = v - nfull * slab
    per_w = nfull // _NW
    rem = nfull - per_w * _NW
    mesh = plsc.VectorSubcoreMesh(core_axis_name="c", subcore_axis_name="s")

    @functools.partial(
        pl.kernel,
        mesh=mesh,
        out_type=jax.ShapeDtypeStruct((v, dpad), jnp.float32),
        scratch_types=[
            pltpu.VMEM((2, d, slab + 1), jnp.float32),
            pltpu.VMEM((2, slab, dpad), jnp.float32),
            pltpu.SemaphoreType.DMA((2,)),
            pltpu.SemaphoreType.DMA((2,)),
        ],
        compiler_params=pltpu.CompilerParams(
            use_tc_tiling_on_sc=True, needs_layout_passes=False
        ),
    )
    def transp(t64_hbm, tail_hbm, out_hbm, in_v, out_v, isem, osem):
        wid = lax.axis_index("s") * _NC + lax.axis_index("c")
        nmine = per_w + jnp.where(wid < rem, 1, 0)

        dd_bases = [
            lax.iota(jnp.int32, _L) + (g * _L) for g in range(d // _L)
        ]

        def transpose_buf(p):
            for vv in range(slab):
                col = jnp.full((_L,), vv, jnp.int32)
                for g in range(d // _L):
                    vals = plsc.load_gather(in_v.at[p], [dd_bases[g], col])
                    out_v[p, vv, pl.ds(g * _L, _L)] = vals

        def start_in(k, p):
            s = k * _NW + wid
            pltpu.async_copy(
                t64_hbm.at[:, pl.ds(s * slab, slab)],
                in_v.at[p, :, pl.ds(0, slab)],
                isem.at[p],
            )

        def wait_in(p):
            pltpu.make_async_copy(
                t64_hbm.at[:, pl.ds(0, slab)],
                in_v.at[p, :, pl.ds(0, slab)],
                isem.at[p],
            ).wait()

        def start_out(k, p):
            s = k * _NW + wid
            pltpu.async_copy(
                out_v.at[p], out_hbm.at[pl.ds(s * slab, slab)], osem.at[p]
            )

        def wait_out(p):
            pltpu.make_async_copy(
                out_v.at[p], out_hbm.at[pl.ds(0, slab)], osem.at[p]
            ).wait()

        @pl.when(nmine > 0)
        def _():
            start_in(0, 0)

        def body(k, carry):
            p = lax.rem(k, 2)
            q = 1 - p
            @pl.when(k + 1 < nmine)
            def _():
                start_in(k + 1, q)
            wait_in(p)
            @pl.when(k >= 2)
            def _():
                wait_out(p)
            transpose_buf(p)
            start_out(k, p)
            return carry

        lax.fori_loop(0, nmine, body, 0, unroll=False)

        @pl.when(nmine >= 2)
        def _():
            wait_out(1 - lax.rem(nmine, 2))
        @pl.when(nmine >= 1)
        def _():
            wait_out(lax.rem(nmine, 2))

        if tail:
            @pl.when(wid == 0)
            def _():
                pltpu.async_copy(
                    tail_hbm, out_v.at[0, pl.ds(0, tail)], isem.at[0]
                )
                pltpu.make_async_copy(
                    tail_hbm, out_v.at[0, pl.ds(0, tail)], isem.at[0]
                ).wait()
                pltpu.async_copy(
                    out_v.at[0, pl.ds(0, tail)],
                    out_hbm.at[pl.ds(nfull * slab, tail)],
                    osem.at[0],
                )
                pltpu.make_async_copy(
                    out_v.at[0, pl.ds(0, tail)],
                    out_hbm.at[pl.ds(0, tail)],
                    osem.at[0],
                ).wait()

    return transp


@functools.lru_cache(maxsize=None)
def _build_gather(bsz, seqlen, d, dpad):
    seq_per_w = bsz // _NW
    per_w = seq_per_w * seqlen
    c0 = 128
    c1 = seqlen - c0
    ngroup = seq_per_w // _NB
    mesh = plsc.VectorSubcoreMesh(core_axis_name="c", subcore_axis_name="s")

    @functools.partial(
        pl.kernel,
        mesh=mesh,
        out_type=jax.ShapeDtypeStruct((bsz, seqlen, dpad), jnp.float32),
        scratch_types=[
            pltpu.VMEM((per_w,), jnp.int32),
            pltpu.VMEM((_NB, seqlen, dpad), jnp.float32),
            pltpu.SemaphoreType.DMA((_NB,)),
            pltpu.SemaphoreType.DMA((_NB,)),
        ],
        compiler_params=pltpu.CompilerParams(use_tc_tiling_on_sc=True),
    )
    def emb(table_hbm, idx_hbm, out_hbm, idx_v, rows_v, gsem, wsem):
        wid = lax.axis_index("s") * _NC + lax.axis_index("c")
        sbase = wid * seq_per_w
        pltpu.sync_copy(idx_hbm.at[pl.ds(sbase * seqlen, per_w)], idx_v)

        def start_gathers(s, b):
            off = s * seqlen
            pltpu.async_copy(
                table_hbm.at[idx_v.at[pl.ds(off, c0)]],
                rows_v.at[b, pl.ds(0, c0)],
                gsem.at[b],
            )
            pltpu.async_copy(
                table_hbm.at[idx_v.at[pl.ds(off + c0, c1)]],
                rows_v.at[b, pl.ds(c0, c1)],
                gsem.at[b],
            )

        def wait_gathers(b):
            pltpu.make_async_copy(
                table_hbm.at[pl.ds(0, seqlen)], rows_v.at[b], gsem.at[b]
            ).wait()

        def start_write(s, b):
            pltpu.async_copy(
                rows_v.at[b], out_hbm.at[sbase + s], wsem.at[b]
            )

        def wait_write(b):
            pltpu.make_async_copy(
                rows_v.at[b], out_hbm.at[sbase], wsem.at[b]
            ).wait()

        for b in range(_NB):
            start_gathers(b, b)

        def outer(g, carry):
            s0 = g * _NB
            for b in range(_NB):
                wait_gathers(b)
                start_write(s0 + b, b)
            for b in range(_NB):
                wait_write(b)
                start_gathers(s0 + _NB + b, b)
            return carry

        lax.fori_loop(0, ngroup - 1, outer, 0)

        s0 = (ngroup - 1) * _NB
        for b in range(_NB):
            wait_gathers(b)
            start_write(s0 + b, b)
        for b in range(_NB):
            wait_write(b)

    return emb


def kernel(table, seq):
    b, l = seq.shape
    v, d = table.shape
    dpad = 128
    nfull = v // 128
    tail_rows = jnp.pad(table[nfull * 128:], ((0, 0), (0, dpad - d)))
    padded = _build_transpose(v, d, dpad)(table.T, tail_rows)
    idx = seq.reshape(-1).astype(jnp.int32)
    out = _build_gather(b, l, d, dpad)(padded, idx)
    return out[:, :, :d]


# speedup vs baseline: 2.3958x; 2.3958x over previous
import functools

import jax
import jax.numpy as jnp
from jax import lax
from jax.experimental import pallas as pl
from jax.experimental.pallas import tpu as pltpu
from jax.experimental.pallas import tpu_sc as plsc

_NC = 2
_NS = 16
_NW = _NC * _NS

_NB = 4


@functools.lru_cache(maxsize=None)
def _build_gather(bsz, seqlen, d, dpad):
    seq_per_w = bsz // _NW
    per_w = seq_per_w * seqlen
    c0 = 128
    c1 = seqlen - c0
    ngroup = seq_per_w // _NB
    mesh = plsc.VectorSubcoreMesh(core_axis_name="c", subcore_axis_name="s")

    @functools.partial(
        pl.kernel,
        mesh=mesh,
        out_type=jax.ShapeDtypeStruct((bsz, seqlen, dpad), jnp.float32),
        scratch_types=[
            pltpu.VMEM((per_w,), jnp.int32),
            pltpu.VMEM((_NB, seqlen, dpad), jnp.float32),
            pltpu.SemaphoreType.DMA((_NB,)),
            pltpu.SemaphoreType.DMA((_NB,)),
        ],
        compiler_params=pltpu.CompilerParams(use_tc_tiling_on_sc=True),
    )
    def emb(table_hbm, idx_hbm, out_hbm, idx_v, rows_v, gsem, wsem):
        wid = lax.axis_index("s") * _NC + lax.axis_index("c")
        sbase = wid * seq_per_w
        pltpu.sync_copy(idx_hbm.at[pl.ds(sbase * seqlen, per_w)], idx_v)

        def start_gathers(s, b):
            off = s * seqlen
            pltpu.async_copy(
                table_hbm.at[idx_v.at[pl.ds(off, c0)]],
                rows_v.at[b, pl.ds(0, c0)],
                gsem.at[b],
            )
            pltpu.async_copy(
                table_hbm.at[idx_v.at[pl.ds(off + c0, c1)]],
                rows_v.at[b, pl.ds(c0, c1)],
                gsem.at[b],
            )

        def wait_gathers(b):
            pltpu.make_async_copy(
                table_hbm.at[pl.ds(0, seqlen)], rows_v.at[b], gsem.at[b]
            ).wait()

        def start_write(s, b):
            pltpu.async_copy(
                rows_v.at[b], out_hbm.at[sbase + s], wsem.at[b]
            )

        def wait_write(b):
            pltpu.make_async_copy(
                rows_v.at[b], out_hbm.at[sbase], wsem.at[b]
            ).wait()

        for b in range(_NB):
            start_gathers(b, b)

        def outer(g, carry):
            s0 = g * _NB
            for b in range(_NB):
                wait_gathers(b)
                start_write(s0 + b, b)
            for b in range(_NB):
                wait_write(b)
                start_gathers(s0 + _NB + b, b)
            return carry

        lax.fori_loop(0, ngroup - 1, outer, 0)

        s0 = (ngroup - 1) * _NB
        for b in range(_NB):
            wait_gathers(b)
            start_write(s0 + b, b)
        for b in range(_NB):
            wait_write(b)

    return emb


def kernel(table, seq):
    b, l = seq.shape
    v, d = table.shape
    dpad = 128
    padded = jnp.pad(table, ((0, 0), (0, dpad - d)))
    idx = seq.reshape(-1).astype(jnp.int32)
    out = _build_gather(b, l, d, dpad)(padded, idx)
    return out[:, :, :d]
